# speedup vs baseline: 18.0212x; 18.0212x over previous
import jax
import jax.numpy as jnp
from jax.experimental import pallas as pl
from jax.experimental.pallas import tpu as pltpu

TILE_N = 512


def _k1_body(qp_ref, pxyz_ref, pp_ref, skip_ref, w1a_ref, w1b_ref,
             h1_ref, stats_ref, g_scr):
    b = pl.program_id(0)
    n = pl.program_id(1)

    @pl.when(n == 0)
    def _():
        g_scr[:] = jnp.dot(w1a_ref[:], pp_ref[0],
                           preferred_element_type=jnp.float32)

    q = qp_ref[0]
    p = pxyz_ref[0]
    sq_s = jnp.sum(q * q, axis=1, keepdims=True)
    sq_p = jnp.sum(p * p, axis=0, keepdims=True)
    inner = jnp.dot(q, p, preferred_element_type=jnp.float32)
    d2 = jnp.maximum(sq_s + sq_p - 2.0 * inner, 0.0)
    dist = jnp.sqrt(d2)
    recip = 1.0 / (dist + 1e-8)
    norm = jnp.sum(recip, axis=1, keepdims=True)

    col = jax.lax.broadcasted_iota(jnp.int32, dist.shape, 1)
    dwork = dist
    wacc = jnp.zeros_like(dist)
    for _ in range(3):
        m = jnp.min(dwork, axis=1, keepdims=True)
        eq = dwork == m
        sel = jnp.min(jnp.where(eq, col, jnp.int32(2 ** 30)), axis=1,
                      keepdims=True)
        onehot = col == sel
        wacc = wacc + jnp.where(onehot, recip, 0.0)
        dwork = jnp.where(onehot, jnp.float32(jnp.inf), dwork)
    wmat = wacc / norm

    part_a = jax.lax.dot_general(g_scr[:], wmat, (((1,), (1,)), ((), ())),
                                 preferred_element_type=jnp.float32)
    part_b = jax.lax.dot_general(w1b_ref[:], skip_ref[0],
                                 (((1,), (0,)), ((), ())),
                                 preferred_element_type=jnp.float32)
    h1 = part_a + part_b
    h1_ref[0] = h1

    @pl.when((b == 0) & (n == 0))
    def _():
        stats_ref[:] = jnp.zeros_like(stats_ref)

    stats_ref[0:1, :] = stats_ref[0:1, :] + jnp.sum(h1, axis=1)[None, :]
    stats_ref[1:2, :] = stats_ref[1:2, :] + jnp.sum(h1 * h1, axis=1)[None, :]


def _k2_body(h1_ref, scale_ref, shift_ref, w2_ref, h2_ref, stats_ref):
    b = pl.program_id(0)
    n = pl.program_id(1)
    g = jnp.maximum(h1_ref[0] * scale_ref[:] + shift_ref[:], 0.0)
    h2 = jax.lax.dot_general(w2_ref[:], g, (((1,), (0,)), ((), ())),
                             preferred_element_type=jnp.float32)
    h2_ref[0] = h2

    @pl.when((b == 0) & (n == 0))
    def _():
        stats_ref[:] = jnp.zeros_like(stats_ref)

    stats_ref[0:1, :] = stats_ref[0:1, :] + jnp.sum(h2, axis=1)[None, :]
    stats_ref[1:2, :] = stats_ref[1:2, :] + jnp.sum(h2 * h2, axis=1)[None, :]


def _k3_body(h2_ref, scale_ref, shift_ref, out_ref):
    out_ref[0] = jnp.maximum(h2_ref[0] * scale_ref[:] + shift_ref[:], 0.0)


def _affine(stats, count, gamma, beta, eps=1e-5):
    mean = stats[0] / count
    var = stats[1] / count - mean * mean
    scale = gamma * jax.lax.rsqrt(var + eps)
    shift = beta - mean * scale
    return scale.reshape(-1, 1), shift.reshape(-1, 1)


def kernel(xyz_prev, xyz_skip, points_prev, points_skip,
           W1, gamma1, beta1, W2, gamma2, beta2):
    B, N_prev, _ = xyz_prev.shape
    N_skip = xyz_skip.shape[1]
    C_prev = points_prev.shape[1]
    C_skip = points_skip.shape[1]
    C_out = W1.shape[0]
    NT = N_skip // TILE_N
    count = jnp.float32(B * N_skip)

    qp = jnp.pad(xyz_skip, ((0, 0), (0, 0), (0, 5)))
    pxyz = jnp.pad(jnp.transpose(xyz_prev, (0, 2, 1)),
                   ((0, 0), (0, 5), (0, 0)))
    w1a = W1[:, :C_prev]
    w1b = W1[:, C_prev:]

    grid = (B, NT)
    cparams = pltpu.CompilerParams(
        dimension_semantics=("arbitrary", "arbitrary"))

    h1, stats1 = pl.pallas_call(
        _k1_body,
        grid=grid,
        in_specs=[
            pl.BlockSpec((1, TILE_N, 8), lambda b, n: (b, n, 0)),
            pl.BlockSpec((1, 8, N_prev), lambda b, n: (b, 0, 0)),
            pl.BlockSpec((1, C_prev, N_prev), lambda b, n: (b, 0, 0)),
            pl.BlockSpec((1, C_skip, TILE_N), lambda b, n: (b, 0, n)),
            pl.BlockSpec((C_out, C_prev), lambda b, n: (0, 0)),
            pl.BlockSpec((C_out, C_skip), lambda b, n: (0, 0)),
        ],
        out_specs=[
            pl.BlockSpec((1, C_out, TILE_N), lambda b, n: (b, 0, n)),
            pl.BlockSpec((8, C_out), lambda b, n: (0, 0)),
        ],
        out_shape=[
            jax.ShapeDtypeStruct((B, C_out, N_skip), jnp.float32),
            jax.ShapeDtypeStruct((8, C_out), jnp.float32),
        ],
        scratch_shapes=[pltpu.VMEM((C_out, N_prev), jnp.float32)],
        compiler_params=cparams,
    )(qp, pxyz, points_prev, points_skip, w1a, w1b)

    scale1, shift1 = _affine(stats1, count, gamma1, beta1)

    h2, stats2 = pl.pallas_call(
        _k2_body,
        grid=grid,
        in_specs=[
            pl.BlockSpec((1, C_out, TILE_N), lambda b, n: (b, 0, n)),
            pl.BlockSpec((C_out, 1), lambda b, n: (0, 0)),
            pl.BlockSpec((C_out, 1), lambda b, n: (0, 0)),
            pl.BlockSpec((C_out, C_out), lambda b, n: (0, 0)),
        ],
        out_specs=[
            pl.BlockSpec((1, C_out, TILE_N), lambda b, n: (b, 0, n)),
            pl.BlockSpec((8, C_out), lambda b, n: (0, 0)),
        ],
        out_shape=[
            jax.ShapeDtypeStruct((B, C_out, N_skip), jnp.float32),
            jax.ShapeDtypeStruct((8, C_out), jnp.float32),
        ],
        compiler_params=cparams,
    )(h1, scale1, shift1, W2)

    scale2, shift2 = _affine(stats2, count, gamma2, beta2)

    out = pl.pallas_call(
        _k3_body,
        grid=grid,
        in_specs=[
            pl.BlockSpec((1, C_out, TILE_N), lambda b, n: (b, 0, n)),
            pl.BlockSpec((C_out, 1), lambda b, n: (0, 0)),
            pl.BlockSpec((C_out, 1), lambda b, n: (0, 0)),
        ],
        out_specs=pl.BlockSpec((1, C_out, TILE_N), lambda b, n: (b, 0, n)),
        out_shape=jax.ShapeDtypeStruct((B, C_out, N_skip), jnp.float32),
        compiler_params=cparams,
    )(h2, scale2, shift2)

    return out


# speedup vs baseline: 19.3946x; 1.0762x over previous
import jax
import jax.numpy as jnp
from jax.experimental import pallas as pl
from jax.experimental.pallas import tpu as pltpu

TILE_N = 512


def _k1_body(qp_ref, pxyz_ref, pp_ref, skip_ref, w1a_ref, w1b_ref,
             h1_ref, stats_ref, g_scr):
    b = pl.program_id(0)
    n = pl.program_id(1)

    @pl.when(n == 0)
    def _():
        g_scr[:] = jnp.dot(w1a_ref[:], pp_ref[0],
                           preferred_element_type=jnp.float32)

    q = qp_ref[0]
    p = pxyz_ref[0]
    sq_s = jnp.sum(q * q, axis=1, keepdims=True)
    sq_p = jnp.sum(p * p, axis=0, keepdims=True)
    inner = jnp.dot(q, p, preferred_element_type=jnp.float32)
    d2 = jnp.maximum(sq_s + sq_p - 2.0 * inner, 1e-30)
    recip = jax.lax.rsqrt(d2)
    norm = jnp.sum(recip, axis=1)

    col = jax.lax.broadcasted_iota(jnp.int32, d2.shape, 1)
    dwork = d2
    selmask = jnp.zeros(d2.shape, jnp.bool_)
    for _ in range(3):
        m = jnp.min(dwork, axis=1, keepdims=True)
        eq = dwork == m
        sel = jnp.min(jnp.where(eq, col, jnp.int32(2 ** 30)), axis=1,
                      keepdims=True)
        onehot = col == sel
        selmask = selmask | onehot
        dwork = jnp.where(onehot, jnp.float32(jnp.inf), dwork)
    wacc = jnp.where(selmask, recip, 0.0)

    part_a = jax.lax.dot_general(g_scr[:], wacc, (((1,), (1,)), ((), ())),
                                 preferred_element_type=jnp.float32)
    part_a = part_a * (1.0 / norm)[None, :]
    part_b = jax.lax.dot_general(w1b_ref[:], skip_ref[0],
                                 (((1,), (0,)), ((), ())),
                                 preferred_element_type=jnp.float32)
    h1 = part_a + part_b
    h1_ref[0] = h1

    @pl.when((b == 0) & (n == 0))
    def _():
        stats_ref[:] = jnp.zeros_like(stats_ref)

    stats_ref[0:1, :] = stats_ref[0:1, :] + jnp.sum(h1, axis=1)[None, :]
    stats_ref[1:2, :] = stats_ref[1:2, :] + jnp.sum(h1 * h1, axis=1)[None, :]


def _k2_body(h1_ref, scale_ref, shift_ref, w2_ref, h2_ref, stats_ref):
    b = pl.program_id(0)
    n = pl.program_id(1)
    g = jnp.maximum(h1_ref[0] * scale_ref[:] + shift_ref[:], 0.0)
    h2 = jax.lax.dot_general(w2_ref[:], g, (((1,), (0,)), ((), ())),
                             preferred_element_type=jnp.float32)
    h2_ref[0] = h2

    @pl.when((b == 0) & (n == 0))
    def _():
        stats_ref[:] = jnp.zeros_like(stats_ref)

    stats_ref[0:1, :] = stats_ref[0:1, :] + jnp.sum(h2, axis=1)[None, :]
    stats_ref[1:2, :] = stats_ref[1:2, :] + jnp.sum(h2 * h2, axis=1)[None, :]


def _k3_body(h2_ref, scale_ref, shift_ref, out_ref):
    out_ref[0] = jnp.maximum(h2_ref[0] * scale_ref[:] + shift_ref[:], 0.0)


def _affine(stats, count, gamma, beta, eps=1e-5):
    mean = stats[0] / count
    var = stats[1] / count - mean * mean
    scale = gamma * jax.lax.rsqrt(var + eps)
    shift = beta - mean * scale
    return scale.reshape(-1, 1), shift.reshape(-1, 1)


def kernel(xyz_prev, xyz_skip, points_prev, points_skip,
           W1, gamma1, beta1, W2, gamma2, beta2):
    B, N_prev, _ = xyz_prev.shape
    N_skip = xyz_skip.shape[1]
    C_prev = points_prev.shape[1]
    C_skip = points_skip.shape[1]
    C_out = W1.shape[0]
    NT = N_skip // TILE_N
    count = jnp.float32(B * N_skip)

    qp = jnp.pad(xyz_skip, ((0, 0), (0, 0), (0, 5)))
    pxyz = jnp.pad(jnp.transpose(xyz_prev, (0, 2, 1)),
                   ((0, 0), (0, 5), (0, 0)))
    w1a = W1[:, :C_prev]
    w1b = W1[:, C_prev:]

    grid = (B, NT)
    cparams = pltpu.CompilerParams(
        dimension_semantics=("arbitrary", "arbitrary"))

    h1, stats1 = pl.pallas_call(
        _k1_body,
        grid=grid,
        in_specs=[
            pl.BlockSpec((1, TILE_N, 8), lambda b, n: (b, n, 0)),
            pl.BlockSpec((1, 8, N_prev), lambda b, n: (b, 0, 0)),
            pl.BlockSpec((1, C_prev, N_prev), lambda b, n: (b, 0, 0)),
            pl.BlockSpec((1, C_skip, TILE_N), lambda b, n: (b, 0, n)),
            pl.BlockSpec((C_out, C_prev), lambda b, n: (0, 0)),
            pl.BlockSpec((C_out, C_skip), lambda b, n: (0, 0)),
        ],
        out_specs=[
            pl.BlockSpec((1, C_out, TILE_N), lambda b, n: (b, 0, n)),
            pl.BlockSpec((8, C_out), lambda b, n: (0, 0)),
        ],
        out_shape=[
            jax.ShapeDtypeStruct((B, C_out, N_skip), jnp.float32),
            jax.ShapeDtypeStruct((8, C_out), jnp.float32),
        ],
        scratch_shapes=[pltpu.VMEM((C_out, N_prev), jnp.float32)],
        compiler_params=cparams,
    )(qp, pxyz, points_prev, points_skip, w1a, w1b)

    scale1, shift1 = _affine(stats1, count, gamma1, beta1)

    h2, stats2 = pl.pallas_call(
        _k2_body,
        grid=grid,
        in_specs=[
            pl.BlockSpec((1, C_out, TILE_N), lambda b, n: (b, 0, n)),
            pl.BlockSpec((C_out, 1), lambda b, n: (0, 0)),
            pl.BlockSpec((C_out, 1), lambda b, n: (0, 0)),
            pl.BlockSpec((C_out, C_out), lambda b, n: (0, 0)),
        ],
        out_specs=[
            pl.BlockSpec((1, C_out, TILE_N), lambda b, n: (b, 0, n)),
            pl.BlockSpec((8, C_out), lambda b, n: (0, 0)),
        ],
        out_shape=[
            jax.ShapeDtypeStruct((B, C_out, N_skip), jnp.float32),
            jax.ShapeDtypeStruct((8, C_out), jnp.float32),
        ],
        compiler_params=cparams,
    )(h1, scale1, shift1, W2)

    scale2, shift2 = _affine(stats2, count, gamma2, beta2)

    out = pl.pallas_call(
        _k3_body,
        grid=grid,
        in_specs=[
            pl.BlockSpec((1, C_out, TILE_N), lambda b, n: (b, 0, n)),
            pl.BlockSpec((C_out, 1), lambda b, n: (0, 0)),
            pl.BlockSpec((C_out, 1), lambda b, n: (0, 0)),
        ],
        out_specs=pl.BlockSpec((1, C_out, TILE_N), lambda b, n: (b, 0, n)),
        out_shape=jax.ShapeDtypeStruct((B, C_out, N_skip), jnp.float32),
        compiler_params=cparams,
    )(h2, scale2, shift2)

    return out


# speedup vs baseline: 28.5090x; 1.4699x over previous
import jax
import jax.numpy as jnp
from jax.experimental import pallas as pl
from jax.experimental.pallas import tpu as pltpu

TILE_N = 1024


def _k1_body(qp_ref, pxyz_ref, pp_ref, skip_ref, w1a_ref, w1b_ref,
             h1_ref, stats_ref, g_scr):
    b = pl.program_id(0)
    n = pl.program_id(1)

    @pl.when(n == 0)
    def _():
        g_scr[:] = jnp.dot(w1a_ref[:], pp_ref[0],
                           preferred_element_type=jnp.float32)

    q = qp_ref[0]
    p = pxyz_ref[0]
    sq_s = jnp.sum(q * q, axis=1, keepdims=True)
    sq_p = jnp.sum(p * p, axis=0, keepdims=True)
    inner = jnp.dot(q, p, preferred_element_type=jnp.float32)
    d2 = jnp.maximum(sq_s + sq_p - 2.0 * inner, 1e-30)
    recip = jax.lax.rsqrt(d2)
    norm = jnp.sum(recip, axis=1)

    col = jax.lax.broadcasted_iota(jnp.int32, d2.shape, 1)
    key = jax.lax.bitcast_convert_type(
        (jax.lax.bitcast_convert_type(d2, jnp.int32) & jnp.int32(~0x3FF))
        | col, jnp.float32)
    selmask = jnp.zeros(d2.shape, jnp.bool_)
    for _ in range(3):
        m = jnp.min(key, axis=1, keepdims=True)
        onehot = key == m
        selmask = selmask | onehot
        key = jnp.where(onehot, jnp.float32(jnp.inf), key)
    wacc = jnp.where(selmask, recip, 0.0)

    part_a = jax.lax.dot_general(g_scr[:], wacc, (((1,), (1,)), ((), ())),
                                 preferred_element_type=jnp.float32)
    part_a = part_a * (1.0 / norm)[None, :]
    part_b = jax.lax.dot_general(w1b_ref[:], skip_ref[0],
                                 (((1,), (0,)), ((), ())),
                                 preferred_element_type=jnp.float32)
    h1 = part_a + part_b
    h1_ref[0] = h1

    @pl.when((b == 0) & (n == 0))
    def _():
        stats_ref[:] = jnp.zeros_like(stats_ref)

    stats_ref[:, 0:1] = stats_ref[:, 0:1] + jnp.sum(h1, axis=1, keepdims=True)
    stats_ref[:, 1:2] = stats_ref[:, 1:2] + jnp.sum(h1 * h1, axis=1,
                                                    keepdims=True)


def _k2_body(h1_ref, scale_ref, shift_ref, w2_ref, h2_ref, stats_ref):
    b = pl.program_id(0)
    n = pl.program_id(1)
    g = jnp.maximum(h1_ref[0] * scale_ref[:] + shift_ref[:], 0.0)
    h2 = jax.lax.dot_general(w2_ref[:], g, (((1,), (0,)), ((), ())),
                             preferred_element_type=jnp.float32)
    h2_ref[0] = h2

    @pl.when((b == 0) & (n == 0))
    def _():
        stats_ref[:] = jnp.zeros_like(stats_ref)

    stats_ref[:, 0:1] = stats_ref[:, 0:1] + jnp.sum(h2, axis=1, keepdims=True)
    stats_ref[:, 1:2] = stats_ref[:, 1:2] + jnp.sum(h2 * h2, axis=1,
                                                    keepdims=True)


def _k3_body(h2_ref, scale_ref, shift_ref, out_ref):
    out_ref[0] = jnp.maximum(h2_ref[0] * scale_ref[:] + shift_ref[:], 0.0)


def _affine(stats, count, gamma, beta, eps=1e-5):
    mean = stats[:, 0] / count
    var = stats[:, 1] / count - mean * mean
    scale = gamma * jax.lax.rsqrt(var + eps)
    shift = beta - mean * scale
    return scale.reshape(-1, 1), shift.reshape(-1, 1)


def kernel(xyz_prev, xyz_skip, points_prev, points_skip,
           W1, gamma1, beta1, W2, gamma2, beta2):
    B, N_prev, _ = xyz_prev.shape
    N_skip = xyz_skip.shape[1]
    C_prev = points_prev.shape[1]
    C_skip = points_skip.shape[1]
    C_out = W1.shape[0]
    NT = N_skip // TILE_N
    count = jnp.float32(B * N_skip)

    qp = jnp.pad(xyz_skip, ((0, 0), (0, 0), (0, 5)))
    pxyz = jnp.pad(jnp.transpose(xyz_prev, (0, 2, 1)),
                   ((0, 0), (0, 5), (0, 0)))
    w1a = W1[:, :C_prev]
    w1b = W1[:, C_prev:]

    grid = (B, NT)
    cparams = pltpu.CompilerParams(
        dimension_semantics=("arbitrary", "arbitrary"))

    h1, stats1 = pl.pallas_call(
        _k1_body,
        grid=grid,
        in_specs=[
            pl.BlockSpec((1, TILE_N, 8), lambda b, n: (b, n, 0)),
            pl.BlockSpec((1, 8, N_prev), lambda b, n: (b, 0, 0)),
            pl.BlockSpec((1, C_prev, N_prev), lambda b, n: (b, 0, 0)),
            pl.BlockSpec((1, C_skip, TILE_N), lambda b, n: (b, 0, n)),
            pl.BlockSpec((C_out, C_prev), lambda b, n: (0, 0)),
            pl.BlockSpec((C_out, C_skip), lambda b, n: (0, 0)),
        ],
        out_specs=[
            pl.BlockSpec((1, C_out, TILE_N), lambda b, n: (b, 0, n)),
            pl.BlockSpec((C_out, 8), lambda b, n: (0, 0)),
        ],
        out_shape=[
            jax.ShapeDtypeStruct((B, C_out, N_skip), jnp.float32),
            jax.ShapeDtypeStruct((C_out, 8), jnp.float32),
        ],
        scratch_shapes=[pltpu.VMEM((C_out, N_prev), jnp.float32)],
        compiler_params=cparams,
    )(qp, pxyz, points_prev, points_skip, w1a, w1b)

    scale1, shift1 = _affine(stats1, count, gamma1, beta1)

    h2, stats2 = pl.pallas_call(
        _k2_body,
        grid=grid,
        in_specs=[
            pl.BlockSpec((1, C_out, TILE_N), lambda b, n: (b, 0, n)),
            pl.BlockSpec((C_out, 1), lambda b, n: (0, 0)),
            pl.BlockSpec((C_out, 1), lambda b, n: (0, 0)),
            pl.BlockSpec((C_out, C_out), lambda b, n: (0, 0)),
        ],
        out_specs=[
            pl.BlockSpec((1, C_out, TILE_N), lambda b, n: (b, 0, n)),
            pl.BlockSpec((C_out, 8), lambda b, n: (0, 0)),
        ],
        out_shape=[
            jax.ShapeDtypeStruct((B, C_out, N_skip), jnp.float32),
            jax.ShapeDtypeStruct((C_out, 8), jnp.float32),
        ],
        compiler_params=cparams,
    )(h1, scale1, shift1, W2)

    scale2, shift2 = _affine(stats2, count, gamma2, beta2)

    out = pl.pallas_call(
        _k3_body,
        grid=grid,
        in_specs=[
            pl.BlockSpec((1, C_out, TILE_N), lambda b, n: (b, 0, n)),
            pl.BlockSpec((C_out, 1), lambda b, n: (0, 0)),
            pl.BlockSpec((C_out, 1), lambda b, n: (0, 0)),
        ],
        out_specs=pl.BlockSpec((1, C_out, TILE_N), lambda b, n: (b, 0, n)),
        out_shape=jax.ShapeDtypeStruct((B, C_out, N_skip), jnp.float32),
        compiler_params=cparams,
    )(h2, scale2, shift2)

    return out


# speedup vs baseline: 31.1481x; 1.0926x over previous
import jax
import jax.numpy as jnp
from jax.experimental import pallas as pl
from jax.experimental.pallas import tpu as pltpu

TILE_N = 1024


def _k1_body(qp_ref, pxyz_ref, pp_ref, skip_ref, w1a_ref, w1b_ref,
             h1_ref, stats_ref, g_scr):
    b = pl.program_id(0)
    n = pl.program_id(1)

    @pl.when(n == 0)
    def _():
        g_scr[:] = jnp.dot(w1a_ref[:], pp_ref[0],
                           preferred_element_type=jnp.float32)

    q = qp_ref[0]
    p = pxyz_ref[0]
    sq_s = jnp.sum(q * q, axis=1, keepdims=True)
    sq_p = jnp.sum(p * p, axis=0, keepdims=True)
    inner = jnp.dot(q * -2.0, p, preferred_element_type=jnp.float32)
    d2 = jnp.maximum(sq_s + sq_p + inner, 1e-30)
    recip = jax.lax.rsqrt(d2)
    norm = jnp.sum(recip, axis=1)

    col = jax.lax.broadcasted_iota(jnp.int32, d2.shape, 1)
    key = jax.lax.bitcast_convert_type(
        (jax.lax.bitcast_convert_type(d2, jnp.int32) & jnp.int32(~0x3FF))
        | col, jnp.float32)
    for _ in range(3):
        m = jnp.min(key, axis=1, keepdims=True)
        key = jnp.where(key == m, jnp.float32(jnp.inf), key)
    wacc = jnp.where(key == jnp.float32(jnp.inf), recip, 0.0)

    part_a = jax.lax.dot_general(g_scr[:], wacc, (((1,), (1,)), ((), ())),
                                 preferred_element_type=jnp.float32)
    part_a = part_a * (1.0 / norm)[None, :]
    part_b = jax.lax.dot_general(w1b_ref[:], skip_ref[0],
                                 (((1,), (0,)), ((), ())),
                                 preferred_element_type=jnp.float32)
    h1 = part_a + part_b
    h1_ref[0] = h1

    @pl.when((b == 0) & (n == 0))
    def _():
        stats_ref[:] = jnp.zeros_like(stats_ref)

    stats_ref[:, 0:1] = stats_ref[:, 0:1] + jnp.sum(h1, axis=1, keepdims=True)
    stats_ref[:, 1:2] = stats_ref[:, 1:2] + jnp.sum(h1 * h1, axis=1,
                                                    keepdims=True)


def _k2_body(h1_ref, scale_ref, shift_ref, w2_ref, h2_ref, stats_ref):
    b = pl.program_id(0)
    n = pl.program_id(1)
    g = jnp.maximum(h1_ref[0] * scale_ref[:] + shift_ref[:], 0.0)
    h2 = jax.lax.dot_general(w2_ref[:], g, (((1,), (0,)), ((), ())),
                             preferred_element_type=jnp.float32)
    h2_ref[0] = h2

    @pl.when((b == 0) & (n == 0))
    def _():
        stats_ref[:] = jnp.zeros_like(stats_ref)

    stats_ref[:, 0:1] = stats_ref[:, 0:1] + jnp.sum(h2, axis=1, keepdims=True)
    stats_ref[:, 1:2] = stats_ref[:, 1:2] + jnp.sum(h2 * h2, axis=1,
                                                    keepdims=True)


def _k3_body(h2_ref, scale_ref, shift_ref, out_ref):
    out_ref[0] = jnp.maximum(h2_ref[0] * scale_ref[:] + shift_ref[:], 0.0)


def _affine(stats, count, gamma, beta, eps=1e-5):
    mean = stats[:, 0] / count
    var = stats[:, 1] / count - mean * mean
    scale = gamma * jax.lax.rsqrt(var + eps)
    shift = beta - mean * scale
    return scale.reshape(-1, 1), shift.reshape(-1, 1)


def kernel(xyz_prev, xyz_skip, points_prev, points_skip,
           W1, gamma1, beta1, W2, gamma2, beta2):
    B, N_prev, _ = xyz_prev.shape
    N_skip = xyz_skip.shape[1]
    C_prev = points_prev.shape[1]
    C_skip = points_skip.shape[1]
    C_out = W1.shape[0]
    NT = N_skip // TILE_N
    count = jnp.float32(B * N_skip)

    qp = jnp.pad(xyz_skip, ((0, 0), (0, 0), (0, 5)))
    pxyz = jnp.pad(jnp.transpose(xyz_prev, (0, 2, 1)),
                   ((0, 0), (0, 5), (0, 0)))
    w1a = W1[:, :C_prev]
    w1b = W1[:, C_prev:]

    grid = (B, NT)
    cparams = pltpu.CompilerParams(
        dimension_semantics=("arbitrary", "arbitrary"))

    h1, stats1 = pl.pallas_call(
        _k1_body,
        grid=grid,
        in_specs=[
            pl.BlockSpec((1, TILE_N, 8), lambda b, n: (b, n, 0)),
            pl.BlockSpec((1, 8, N_prev), lambda b, n: (b, 0, 0)),
            pl.BlockSpec((1, C_prev, N_prev), lambda b, n: (b, 0, 0)),
            pl.BlockSpec((1, C_skip, TILE_N), lambda b, n: (b, 0, n)),
            pl.BlockSpec((C_out, C_prev), lambda b, n: (0, 0)),
            pl.BlockSpec((C_out, C_skip), lambda b, n: (0, 0)),
        ],
        out_specs=[
            pl.BlockSpec((1, C_out, TILE_N), lambda b, n: (b, 0, n)),
            pl.BlockSpec((C_out, 8), lambda b, n: (0, 0)),
        ],
        out_shape=[
            jax.ShapeDtypeStruct((B, C_out, N_skip), jnp.float32),
            jax.ShapeDtypeStruct((C_out, 8), jnp.float32),
        ],
        scratch_shapes=[pltpu.VMEM((C_out, N_prev), jnp.float32)],
        compiler_params=cparams,
    )(qp, pxyz, points_prev, points_skip, w1a, w1b)

    scale1, shift1 = _affine(stats1, count, gamma1, beta1)

    h2, stats2 = pl.pallas_call(
        _k2_body,
        grid=grid,
        in_specs=[
            pl.BlockSpec((1, C_out, TILE_N), lambda b, n: (b, 0, n)),
            pl.BlockSpec((C_out, 1), lambda b, n: (0, 0)),
            pl.BlockSpec((C_out, 1), lambda b, n: (0, 0)),
            pl.BlockSpec((C_out, C_out), lambda b, n: (0, 0)),
        ],
        out_specs=[
            pl.BlockSpec((1, C_out, TILE_N), lambda b, n: (b, 0, n)),
            pl.BlockSpec((C_out, 8), lambda b, n: (0, 0)),
        ],
        out_shape=[
            jax.ShapeDtypeStruct((B, C_out, N_skip), jnp.float32),
            jax.ShapeDtypeStruct((C_out, 8), jnp.float32),
        ],
        compiler_params=cparams,
    )(h1, scale1, shift1, W2)

    scale2, shift2 = _affine(stats2, count, gamma2, beta2)

    out = pl.pallas_call(
        _k3_body,
        grid=grid,
        in_specs=[
            pl.BlockSpec((1, C_out, TILE_N), lambda b, n: (b, 0, n)),
            pl.BlockSpec((C_out, 1), lambda b, n: (0, 0)),
            pl.BlockSpec((C_out, 1), lambda b, n: (0, 0)),
        ],
        out_specs=pl.BlockSpec((1, C_out, TILE_N), lambda b, n: (b, 0, n)),
        out_shape=jax.ShapeDtypeStruct((B, C_out, N_skip), jnp.float32),
        compiler_params=cparams,
    )(h2, scale2, shift2)

    return out


# speedup vs baseline: 36.4104x; 1.1689x over previous
import jax
import jax.numpy as jnp
from jax.experimental import pallas as pl
from jax.experimental.pallas import tpu as pltpu

TILE_N = 1024


def _body(qp_ref, pxyz_ref, pp_ref, skip_ref, w1a_ref, w1b_ref, w2_ref,
          g1_ref, b1_ref, g2_ref, b2_ref, out_ref,
          h_scr, g_scr, stats_scr, aff_scr, *, nt, count):
    p = pl.program_id(0)
    b = pl.program_id(1)
    n = pl.program_id(2)
    i = b * nt + n
    eps = 1e-5

    @pl.when((p == 0) & (b == 0) & (n == 0))
    def _():
        stats_scr[:] = jnp.zeros_like(stats_scr)

    @pl.when((p == 0) & (n == 0))
    def _():
        g_scr[:] = jnp.dot(w1a_ref[:], pp_ref[0],
                           preferred_element_type=jnp.float32)

    @pl.when(p == 0)
    def _():
        q = qp_ref[0]
        pk = pxyz_ref[0]
        sq_s = jnp.sum(q * q, axis=1, keepdims=True)
        sq_p = jnp.sum(pk * pk, axis=0, keepdims=True)
        inner = jnp.dot(q * -2.0, pk, preferred_element_type=jnp.float32)
        d2 = jnp.maximum(sq_s + sq_p + inner, 1e-30)
        recip = jax.lax.rsqrt(d2)
        norm = jnp.sum(recip, axis=1)

        col = jax.lax.broadcasted_iota(jnp.int32, d2.shape, 1)
        key = jax.lax.bitcast_convert_type(
            (jax.lax.bitcast_convert_type(d2, jnp.int32)
             & jnp.int32(~0x3FF)) | col, jnp.float32)
        for _ in range(3):
            m = jnp.min(key, axis=1, keepdims=True)
            key = jnp.where(key == m, jnp.float32(jnp.inf), key)
        wacc = jnp.where(key == jnp.float32(jnp.inf), recip, 0.0)

        part_a = jax.lax.dot_general(g_scr[:], wacc,
                                     (((1,), (1,)), ((), ())),
                                     preferred_element_type=jnp.float32)
        part_a = part_a * (1.0 / norm)[None, :]
        part_b = jax.lax.dot_general(w1b_ref[:], skip_ref[0],
                                     (((1,), (0,)), ((), ())),
                                     preferred_element_type=jnp.float32)
        h1 = part_a + part_b
        h_scr[i] = h1
        stats_scr[:, 0:1] = stats_scr[:, 0:1] + jnp.sum(h1, axis=1,
                                                        keepdims=True)
        stats_scr[:, 1:2] = stats_scr[:, 1:2] + jnp.sum(h1 * h1, axis=1,
                                                        keepdims=True)

    @pl.when((p == 1) & (b == 0) & (n == 0))
    def _():
        mean = stats_scr[:, 0:1] * (1.0 / count)
        var = stats_scr[:, 1:2] * (1.0 / count) - mean * mean
        scale = g1_ref[:] * jax.lax.rsqrt(var + eps)
        aff_scr[:, 0:1] = scale
        aff_scr[:, 1:2] = b1_ref[:] - mean * scale

    @pl.when(p == 1)
    def _():
        g = jnp.maximum(h_scr[i] * aff_scr[:, 0:1] + aff_scr[:, 1:2], 0.0)
        h2 = jax.lax.dot_general(w2_ref[:], g, (((1,), (0,)), ((), ())),
                                 preferred_element_type=jnp.float32)
        h_scr[i] = h2
        stats_scr[:, 2:3] = stats_scr[:, 2:3] + jnp.sum(h2, axis=1,
                                                        keepdims=True)
        stats_scr[:, 3:4] = stats_scr[:, 3:4] + jnp.sum(h2 * h2, axis=1,
                                                        keepdims=True)

    @pl.when((p == 2) & (b == 0) & (n == 0))
    def _():
        mean = stats_scr[:, 2:3] * (1.0 / count)
        var = stats_scr[:, 3:4] * (1.0 / count) - mean * mean
        scale = g2_ref[:] * jax.lax.rsqrt(var + eps)
        aff_scr[:, 2:3] = scale
        aff_scr[:, 3:4] = b2_ref[:] - mean * scale

    @pl.when(p == 2)
    def _():
        out_ref[0] = jnp.maximum(
            h_scr[i] * aff_scr[:, 2:3] + aff_scr[:, 3:4], 0.0)


def kernel(xyz_prev, xyz_skip, points_prev, points_skip,
           W1, gamma1, beta1, W2, gamma2, beta2):
    import functools
    B, N_prev, _ = xyz_prev.shape
    N_skip = xyz_skip.shape[1]
    C_prev = points_prev.shape[1]
    C_skip = points_skip.shape[1]
    C_out = W1.shape[0]
    NT = N_skip // TILE_N
    count = float(B * N_skip)

    qp = jnp.pad(xyz_skip, ((0, 0), (0, 0), (0, 5)))
    pxyz = jnp.pad(jnp.transpose(xyz_prev, (0, 2, 1)),
                   ((0, 0), (0, 5), (0, 0)))
    w1a = W1[:, :C_prev]
    w1b = W1[:, C_prev:]
    g1 = gamma1.reshape(-1, 1)
    b1 = beta1.reshape(-1, 1)
    g2 = gamma2.reshape(-1, 1)
    b2 = beta2.reshape(-1, 1)

    out = pl.pallas_call(
        functools.partial(_body, nt=NT, count=count),
        grid=(3, B, NT),
        in_specs=[
            pl.BlockSpec((1, TILE_N, 8),
                         lambda p, b, n: (jnp.where(p == 0, b, 0),
                                          jnp.where(p == 0, n, 0), 0)),
            pl.BlockSpec((1, 8, N_prev),
                         lambda p, b, n: (jnp.where(p == 0, b, 0), 0, 0)),
            pl.BlockSpec((1, C_prev, N_prev),
                         lambda p, b, n: (jnp.where(p == 0, b, 0), 0, 0)),
            pl.BlockSpec((1, C_skip, TILE_N),
                         lambda p, b, n: (jnp.where(p == 0, b, 0), 0,
                                          jnp.where(p == 0, n, 0))),
            pl.BlockSpec((C_out, C_prev), lambda p, b, n: (0, 0)),
            pl.BlockSpec((C_out, C_skip), lambda p, b, n: (0, 0)),
            pl.BlockSpec((C_out, C_out), lambda p, b, n: (0, 0)),
            pl.BlockSpec((C_out, 1), lambda p, b, n: (0, 0)),
            pl.BlockSpec((C_out, 1), lambda p, b, n: (0, 0)),
            pl.BlockSpec((C_out, 1), lambda p, b, n: (0, 0)),
            pl.BlockSpec((C_out, 1), lambda p, b, n: (0, 0)),
        ],
        out_specs=pl.BlockSpec(
            (1, C_out, TILE_N),
            lambda p, b, n: (jnp.where(p == 2, b, 0), 0,
                             jnp.where(p == 2, n, 0))),
        out_shape=jax.ShapeDtypeStruct((B, C_out, N_skip), jnp.float32),
        scratch_shapes=[
            pltpu.VMEM((B * NT, C_out, TILE_N), jnp.float32),
            pltpu.VMEM((C_out, N_prev), jnp.float32),
            pltpu.VMEM((C_out, 8), jnp.float32),
            pltpu.VMEM((C_out, 8), jnp.float32),
        ],
        compiler_params=pltpu.CompilerParams(
            dimension_semantics=("arbitrary", "arbitrary", "arbitrary")),
    )(qp, pxyz, points_prev, points_skip, w1a, w1b, W2, g1, b1, g2, b2)

    return out


# speedup vs baseline: 37.3844x; 1.0267x over previous
import jax
import jax.numpy as jnp
from jax.experimental import pallas as pl
from jax.experimental.pallas import tpu as pltpu

TILE_N = 1024


def _body(qp_ref, pxyz_ref, pp_ref, skip_ref, w1a_ref, w1b_ref, w2_ref,
          g1_ref, b1_ref, g2_ref, b2_ref, out_ref,
          h_scr, g_scr, stats_scr, aff_scr, col_scr, *, nt, count):
    p = pl.program_id(0)
    b = pl.program_id(1)
    n = pl.program_id(2)
    i = b * nt + n
    eps = 1e-5

    @pl.when((p == 0) & (b == 0) & (n == 0))
    def _():
        stats_scr[:] = jnp.zeros_like(stats_scr)
        col_scr[:] = jax.lax.broadcasted_iota(jnp.int32, col_scr.shape, 1)

    @pl.when((p == 0) & (n == 0))
    def _():
        g_scr[:] = jnp.dot(w1a_ref[:], pp_ref[0],
                           preferred_element_type=jnp.float32)

    @pl.when(p == 0)
    def _():
        q = qp_ref[0]
        pk = pxyz_ref[0]
        sq_s = jnp.sum(q * q, axis=1, keepdims=True)
        sq_p = jnp.sum(pk * pk, axis=0, keepdims=True)
        inner = jnp.dot(q * -2.0, pk, preferred_element_type=jnp.float32)
        d2 = jnp.maximum(sq_s + sq_p + inner, 1e-30)
        recip = jax.lax.rsqrt(d2)
        norm = jnp.sum(recip, axis=1)

        key = jax.lax.bitcast_convert_type(
            (jax.lax.bitcast_convert_type(d2, jnp.int32)
             & jnp.int32(~0x3FF)) | col_scr[:], jnp.float32)
        inf = jnp.float32(jnp.inf)
        v1 = key[:, 0:128]
        v2 = jnp.full_like(v1, inf)
        v3 = jnp.full_like(v1, inf)
        for c in range(1, 8):
            x = key[:, c * 128:(c + 1) * 128]
            hi = jnp.maximum(v1, x)
            v1 = jnp.minimum(v1, x)
            hi2 = jnp.maximum(v2, hi)
            v2 = jnp.minimum(v2, hi)
            v3 = jnp.minimum(v3, hi2)
        m1 = jnp.min(v1, axis=1, keepdims=True)
        v1 = jnp.where(v1 == m1, inf, v1)
        m2 = jnp.minimum(jnp.min(v1, axis=1, keepdims=True),
                         jnp.min(v2, axis=1, keepdims=True))
        v1 = jnp.where(v1 == m2, inf, v1)
        v2 = jnp.where(v2 == m2, inf, v2)
        m3 = jnp.minimum(
            jnp.minimum(jnp.min(v1, axis=1, keepdims=True),
                        jnp.min(v2, axis=1, keepdims=True)),
            jnp.min(v3, axis=1, keepdims=True))
        wacc = jnp.where(key <= m3, recip, 0.0)

        part_a = jax.lax.dot_general(g_scr[:], wacc,
                                     (((1,), (1,)), ((), ())),
                                     preferred_element_type=jnp.float32)
        part_a = part_a * (1.0 / norm)[None, :]
        part_b = jax.lax.dot_general(w1b_ref[:], skip_ref[0],
                                     (((1,), (0,)), ((), ())),
                                     preferred_element_type=jnp.float32)
        h1 = part_a + part_b
        h_scr[i] = h1
        stats_scr[:, 0:1] = stats_scr[:, 0:1] + jnp.sum(h1, axis=1,
                                                        keepdims=True)
        stats_scr[:, 1:2] = stats_scr[:, 1:2] + jnp.sum(h1 * h1, axis=1,
                                                        keepdims=True)

    @pl.when((p == 1) & (b == 0) & (n == 0))
    def _():
        mean = stats_scr[:, 0:1] * (1.0 / count)
        var = stats_scr[:, 1:2] * (1.0 / count) - mean * mean
        scale = g1_ref[:] * jax.lax.rsqrt(var + eps)
        aff_scr[:, 0:1] = scale
        aff_scr[:, 1:2] = b1_ref[:] - mean * scale

    @pl.when(p == 1)
    def _():
        g = jnp.maximum(h_scr[i] * aff_scr[:, 0:1] + aff_scr[:, 1:2], 0.0)
        h2 = jax.lax.dot_general(w2_ref[:], g, (((1,), (0,)), ((), ())),
                                 preferred_element_type=jnp.float32)
        h_scr[i] = h2
        stats_scr[:, 2:3] = stats_scr[:, 2:3] + jnp.sum(h2, axis=1,
                                                        keepdims=True)
        stats_scr[:, 3:4] = stats_scr[:, 3:4] + jnp.sum(h2 * h2, axis=1,
                                                        keepdims=True)

    @pl.when((p == 2) & (b == 0) & (n == 0))
    def _():
        mean = stats_scr[:, 2:3] * (1.0 / count)
        var = stats_scr[:, 3:4] * (1.0 / count) - mean * mean
        scale = g2_ref[:] * jax.lax.rsqrt(var + eps)
        aff_scr[:, 2:3] = scale
        aff_scr[:, 3:4] = b2_ref[:] - mean * scale

    @pl.when(p == 2)
    def _():
        out_ref[0] = jnp.maximum(
            h_scr[i] * aff_scr[:, 2:3] + aff_scr[:, 3:4], 0.0)


def kernel(xyz_prev, xyz_skip, points_prev, points_skip,
           W1, gamma1, beta1, W2, gamma2, beta2):
    import functools
    B, N_prev, _ = xyz_prev.shape
    N_skip = xyz_skip.shape[1]
    C_prev = points_prev.shape[1]
    C_skip = points_skip.shape[1]
    C_out = W1.shape[0]
    NT = N_skip // TILE_N
    count = float(B * N_skip)

    qp = jnp.pad(xyz_skip, ((0, 0), (0, 0), (0, 5)))
    pxyz = jnp.pad(jnp.transpose(xyz_prev, (0, 2, 1)),
                   ((0, 0), (0, 5), (0, 0)))
    w1a = W1[:, :C_prev]
    w1b = W1[:, C_prev:]
    g1 = gamma1.reshape(-1, 1)
    b1 = beta1.reshape(-1, 1)
    g2 = gamma2.reshape(-1, 1)
    b2 = beta2.reshape(-1, 1)

    out = pl.pallas_call(
        functools.partial(_body, nt=NT, count=count),
        grid=(3, B, NT),
        in_specs=[
            pl.BlockSpec((1, TILE_N, 8),
                         lambda p, b, n: (jnp.where(p == 0, b, 0),
                                          jnp.where(p == 0, n, 0), 0)),
            pl.BlockSpec((1, 8, N_prev),
                         lambda p, b, n: (jnp.where(p == 0, b, 0), 0, 0)),
            pl.BlockSpec((1, C_prev, N_prev),
                         lambda p, b, n: (jnp.where(p == 0, b, 0), 0, 0)),
            pl.BlockSpec((1, C_skip, TILE_N),
                         lambda p, b, n: (jnp.where(p == 0, b, 0), 0,
                                          jnp.where(p == 0, n, 0))),
            pl.BlockSpec((C_out, C_prev), lambda p, b, n: (0, 0)),
            pl.BlockSpec((C_out, C_skip), lambda p, b, n: (0, 0)),
            pl.BlockSpec((C_out, C_out), lambda p, b, n: (0, 0)),
            pl.BlockSpec((C_out, 1), lambda p, b, n: (0, 0)),
            pl.BlockSpec((C_out, 1), lambda p, b, n: (0, 0)),
            pl.BlockSpec((C_out, 1), lambda p, b, n: (0, 0)),
            pl.BlockSpec((C_out, 1), lambda p, b, n: (0, 0)),
        ],
        out_specs=pl.BlockSpec(
            (1, C_out, TILE_N),
            lambda p, b, n: (jnp.where(p == 2, b, 0), 0,
                             jnp.where(p == 2, n, 0))),
        out_shape=jax.ShapeDtypeStruct((B, C_out, N_skip), jnp.float32),
        scratch_shapes=[
            pltpu.VMEM((B * NT, C_out, TILE_N), jnp.float32),
            pltpu.VMEM((C_out, N_prev), jnp.float32),
            pltpu.VMEM((C_out, 8), jnp.float32),
            pltpu.VMEM((C_out, 8), jnp.float32),
            pltpu.VMEM((TILE_N, N_prev), jnp.int32),
        ],
        compiler_params=pltpu.CompilerParams(
            dimension_semantics=("arbitrary", "arbitrary", "arbitrary")),
    )(qp, pxyz, points_prev, points_skip, w1a, w1b, W2, g1, b1, g2, b2)

    return out


# speedup vs baseline: 40.1619x; 1.0743x over previous
import functools

import jax
import jax.numpy as jnp
from jax.experimental import pallas as pl
from jax.experimental.pallas import tpu as pltpu

TILE_N = 1024
TILE_W = 2048


def _body(qp_ref, pxyz_ref, pp_ref, skip_ref, w1a_ref, w1b_ref, w2_ref,
          g1_ref, b1_ref, g2_ref, b2_ref, out_ref,
          h_scr, g_scr, stats_scr, aff_scr, col_scr, *,
          nt, nw, n_ph0, n_ph1, count):
    t = pl.program_id(0)
    eps = 1e-5

    @pl.when(t == 0)
    def _():
        stats_scr[:] = jnp.zeros_like(stats_scr)
        col_scr[:] = jax.lax.broadcasted_iota(jnp.int32, col_scr.shape, 1)

    @pl.when((t < n_ph0) & (t % nt == 0))
    def _():
        g_scr[:] = jnp.dot(w1a_ref[:], pp_ref[0],
                           preferred_element_type=jnp.float32)

    @pl.when(t < n_ph0)
    def _():
        q = qp_ref[0]
        pk = pxyz_ref[0]
        sq_s = jnp.sum(q * q, axis=1, keepdims=True)
        sq_p = jnp.sum(pk * pk, axis=0, keepdims=True)
        inner = jnp.dot(q * -2.0, pk, preferred_element_type=jnp.float32)
        d2 = jnp.maximum(sq_s + sq_p + inner, 1e-30)
        recip = jax.lax.rsqrt(d2)
        norm = jnp.sum(recip, axis=1)

        key = jax.lax.bitcast_convert_type(
            (jax.lax.bitcast_convert_type(d2, jnp.int32)
             & jnp.int32(~0x3FF)) | col_scr[:], jnp.float32)
        inf = jnp.float32(jnp.inf)
        v1 = key[:, 0:128]
        v2 = jnp.full_like(v1, inf)
        v3 = jnp.full_like(v1, inf)
        for c in range(1, 8):
            x = key[:, c * 128:(c + 1) * 128]
            hi = jnp.maximum(v1, x)
            v1 = jnp.minimum(v1, x)
            hi2 = jnp.maximum(v2, hi)
            v2 = jnp.minimum(v2, hi)
            v3 = jnp.minimum(v3, hi2)
        m1 = jnp.min(v1, axis=1, keepdims=True)
        v1 = jnp.where(v1 == m1, inf, v1)
        m2 = jnp.minimum(jnp.min(v1, axis=1, keepdims=True),
                         jnp.min(v2, axis=1, keepdims=True))
        v1 = jnp.where(v1 == m2, inf, v1)
        v2 = jnp.where(v2 == m2, inf, v2)
        m3 = jnp.minimum(
            jnp.minimum(jnp.min(v1, axis=1, keepdims=True),
                        jnp.min(v2, axis=1, keepdims=True)),
            jnp.min(v3, axis=1, keepdims=True))
        wacc = jnp.where(key <= m3, recip, 0.0)

        part_a = jax.lax.dot_general(g_scr[:], wacc,
                                     (((1,), (1,)), ((), ())),
                                     preferred_element_type=jnp.float32)
        part_a = part_a * (1.0 / norm)[None, :]
        part_b = jax.lax.dot_general(w1b_ref[:], skip_ref[0],
                                     (((1,), (0,)), ((), ())),
                                     preferred_element_type=jnp.float32)
        h1 = part_a + part_b
        h_scr[t // nt, :, pl.ds((t % nt) * TILE_N, TILE_N)] = h1
        stats_scr[:, 0:1] = stats_scr[:, 0:1] + jnp.sum(h1, axis=1,
                                                        keepdims=True)
        stats_scr[:, 1:2] = stats_scr[:, 1:2] + jnp.sum(h1 * h1, axis=1,
                                                        keepdims=True)

    @pl.when(t == n_ph0)
    def _():
        mean = stats_scr[:, 0:1] * (1.0 / count)
        var = stats_scr[:, 1:2] * (1.0 / count) - mean * mean
        scale = g1_ref[:] * jax.lax.rsqrt(var + eps)
        aff_scr[:, 0:1] = scale
        aff_scr[:, 1:2] = b1_ref[:] - mean * scale

    @pl.when((t >= n_ph0) & (t < n_ph0 + n_ph1))
    def _():
        j = t - n_ph0
        bb = j // nw
        off = (j % nw) * TILE_W
        h1 = h_scr[bb, :, pl.ds(off, TILE_W)]
        g = jnp.maximum(h1 * aff_scr[:, 0:1] + aff_scr[:, 1:2], 0.0)
        h2 = jax.lax.dot_general(w2_ref[:], g, (((1,), (0,)), ((), ())),
                                 preferred_element_type=jnp.float32)
        h_scr[bb, :, pl.ds(off, TILE_W)] = h2
        stats_scr[:, 2:3] = stats_scr[:, 2:3] + jnp.sum(h2, axis=1,
                                                        keepdims=True)
        stats_scr[:, 3:4] = stats_scr[:, 3:4] + jnp.sum(h2 * h2, axis=1,
                                                        keepdims=True)

    @pl.when(t == n_ph0 + n_ph1)
    def _():
        mean = stats_scr[:, 2:3] * (1.0 / count)
        var = stats_scr[:, 3:4] * (1.0 / count) - mean * mean
        scale = g2_ref[:] * jax.lax.rsqrt(var + eps)
        aff_scr[:, 2:3] = scale
        aff_scr[:, 3:4] = b2_ref[:] - mean * scale

    @pl.when(t >= n_ph0 + n_ph1)
    def _():
        j = t - n_ph0 - n_ph1
        bb = j // nw
        off = (j % nw) * TILE_W
        out_ref[0] = jnp.maximum(
            h_scr[bb, :, pl.ds(off, TILE_W)] * aff_scr[:, 2:3]
            + aff_scr[:, 3:4], 0.0)


def kernel(xyz_prev, xyz_skip, points_prev, points_skip,
           W1, gamma1, beta1, W2, gamma2, beta2):
    B, N_prev, _ = xyz_prev.shape
    N_skip = xyz_skip.shape[1]
    C_prev = points_prev.shape[1]
    C_skip = points_skip.shape[1]
    C_out = W1.shape[0]
    NT = N_skip // TILE_N
    NW = N_skip // TILE_W
    N_PH0 = B * NT
    N_PH1 = B * NW
    count = float(B * N_skip)

    qp = jnp.pad(xyz_skip, ((0, 0), (0, 0), (0, 5)))
    pxyz = jnp.pad(jnp.transpose(xyz_prev, (0, 2, 1)),
                   ((0, 0), (0, 5), (0, 0)))
    w1a = W1[:, :C_prev]
    w1b = W1[:, C_prev:]
    g1 = gamma1.reshape(-1, 1)
    b1 = beta1.reshape(-1, 1)
    g2 = gamma2.reshape(-1, 1)
    b2 = beta2.reshape(-1, 1)

    def bmap(t):
        return jnp.where(t < N_PH0, t // NT, 0)

    def nmap(t):
        return jnp.where(t < N_PH0, t % NT, 0)

    out = pl.pallas_call(
        functools.partial(_body, nt=NT, nw=NW, n_ph0=N_PH0, n_ph1=N_PH1,
                          count=count),
        grid=(N_PH0 + 2 * N_PH1,),
        in_specs=[
            pl.BlockSpec((1, TILE_N, 8), lambda t: (bmap(t), nmap(t), 0)),
            pl.BlockSpec((1, 8, N_prev), lambda t: (bmap(t), 0, 0)),
            pl.BlockSpec((1, C_prev, N_prev), lambda t: (bmap(t), 0, 0)),
            pl.BlockSpec((1, C_skip, TILE_N),
                         lambda t: (bmap(t), 0, nmap(t))),
            pl.BlockSpec((C_out, C_prev), lambda t: (0, 0)),
            pl.BlockSpec((C_out, C_skip), lambda t: (0, 0)),
            pl.BlockSpec((C_out, C_out), lambda t: (0, 0)),
            pl.BlockSpec((C_out, 1), lambda t: (0, 0)),
            pl.BlockSpec((C_out, 1), lambda t: (0, 0)),
            pl.BlockSpec((C_out, 1), lambda t: (0, 0)),
            pl.BlockSpec((C_out, 1), lambda t: (0, 0)),
        ],
        out_specs=pl.BlockSpec(
            (1, C_out, TILE_W),
            lambda t: (jnp.where(t >= N_PH0 + N_PH1,
                                 (t - N_PH0 - N_PH1) // NW, 0), 0,
                       jnp.where(t >= N_PH0 + N_PH1,
                                 (t - N_PH0 - N_PH1) % NW, 0))),
        out_shape=jax.ShapeDtypeStruct((B, C_out, N_skip), jnp.float32),
        scratch_shapes=[
            pltpu.VMEM((B, C_out, N_skip), jnp.float32),
            pltpu.VMEM((C_out, N_prev), jnp.float32),
            pltpu.VMEM((C_out, 8), jnp.float32),
            pltpu.VMEM((C_out, 8), jnp.float32),
            pltpu.VMEM((TILE_N, N_prev), jnp.int32),
        ],
        compiler_params=pltpu.CompilerParams(
            dimension_semantics=("arbitrary",)),
    )(qp, pxyz, points_prev, points_skip, w1a, w1b, W2, g1, b1, g2, b2)

    return out
